# baseline (speedup 1.0000x reference)
import jax
import jax.numpy as jnp
from jax import lax
from jax.experimental import pallas as pl
from jax.experimental.pallas import tpu as pltpu

_EPS = 1e-12
_BS = 2048


def _body(x_ref, t_ref, g_ref, b_ref, o_ref):
    emb = x_ref[...] + t_ref[...]
    inv_d = 1.0 / emb.shape[-1]
    mean = jnp.sum(emb, axis=-1, keepdims=True) * inv_d
    var = jnp.sum(emb * emb, axis=-1, keepdims=True) * inv_d - mean * mean
    inv = lax.rsqrt(var + _EPS)
    o_ref[...] = (emb - mean) * (inv * g_ref[...]) + b_ref[...]


def kernel(x, table, gamma, beta):
    B, S, D = x.shape
    bs = _BS if S % _BS == 0 else S
    grid = (S // bs, B)
    return pl.pallas_call(
        _body,
        grid=grid,
        in_specs=[
            pl.BlockSpec((1, bs, D), lambda i, b: (b, i, 0)),
            pl.BlockSpec((bs, D), lambda i, b: (i, 0)),
            pl.BlockSpec((1, D), lambda i, b: (0, 0)),
            pl.BlockSpec((1, D), lambda i, b: (0, 0)),
        ],
        out_specs=pl.BlockSpec((1, bs, D), lambda i, b: (b, i, 0)),
        out_shape=jax.ShapeDtypeStruct((B, S, D), x.dtype),
        compiler_params=pltpu.CompilerParams(
            dimension_semantics=("arbitrary", "arbitrary"),
        ),
    )(x, table[:S], gamma.reshape(1, D), beta.reshape(1, D))


# speedup vs baseline: 1.0011x; 1.0011x over previous
import jax
import jax.numpy as jnp
from jax import lax
from jax.experimental import pallas as pl
from jax.experimental.pallas import tpu as pltpu

_EPS = 1e-12
_BS = 2048


def _body(x_ref, t_ref, g_ref, b_ref, o_ref):
    emb = x_ref[...] + t_ref[...]
    inv_d = 1.0 / emb.shape[-1]
    mean = jnp.sum(emb, axis=-1, keepdims=True) * inv_d
    var = jnp.sum(emb * emb, axis=-1, keepdims=True) * inv_d - mean * mean
    inv = lax.rsqrt(var + _EPS)
    o_ref[...] = (emb - mean) * (inv * g_ref[...]) + b_ref[...]


def kernel(x, table, gamma, beta):
    B, S, D = x.shape
    bs = _BS if S % _BS == 0 else S
    grid = (S // bs, B)
    return pl.pallas_call(
        _body,
        grid=grid,
        in_specs=[
            pl.BlockSpec((1, bs, D), lambda i, b: (b, i, 0)),
            pl.BlockSpec((bs, D), lambda i, b: (i, 0)),
            pl.BlockSpec((1, D), lambda i, b: (0, 0)),
            pl.BlockSpec((1, D), lambda i, b: (0, 0)),
        ],
        out_specs=pl.BlockSpec((1, bs, D), lambda i, b: (b, i, 0)),
        out_shape=jax.ShapeDtypeStruct((B, S, D), x.dtype),
        compiler_params=pltpu.CompilerParams(
            dimension_semantics=("parallel", "parallel"),
        ),
    )(x, table[:S], gamma.reshape(1, D), beta.reshape(1, D))
